# Initial kernel scaffold; baseline (speedup 1.0000x reference)
import jax
import jax.numpy as jnp
from jax.experimental import pallas as pl


def kernel(x, edge_index, W1, b1, W2, b2, W_out, b_out):
    raise NotImplementedError("write your pallas kernel here")


# speedup vs baseline: 262.5606x; 262.5606x over previous
import functools

import jax
import jax.numpy as jnp
from jax import lax
from jax.experimental import pallas as pl
from jax.experimental.pallas import tpu as pltpu
from jax.experimental.pallas import tpu_sc as plsc

N_NODES = 100000
N_EDGES = 6400000
NC, NS = 2, 16
NW = NC * NS
N_PAD = 102400
ROWS = N_PAD // 128
EPW = N_EDGES // NW
CH = 4000
N_CHUNKS = EPW // CH
TILE_N = N_PAD // NS

_mesh = plsc.VectorSubcoreMesh(
    core_axis_name="c", subcore_axis_name="s", num_cores=NC, num_subcores=NS
)


@functools.partial(
    pl.kernel,
    out_type=jax.ShapeDtypeStruct((NC, N_PAD), jnp.float32),
    mesh=_mesh,
    scratch_types=[
        pltpu.VMEM((CH,), jnp.int32),
        pltpu.VMEM((CH,), jnp.float32),
        pltpu.VMEM_SHARED((N_PAD,), jnp.float32),
    ],
)
def _hist(dst_hbm, zeros_hbm, ones_hbm, out_hbm, idx_v, ones_v, acc_sh):
    c = lax.axis_index("c")
    s = lax.axis_index("s")

    @pl.when(s == 0)
    def _():
        pltpu.sync_copy(zeros_hbm, acc_sh)

    pltpu.sync_copy(ones_hbm, ones_v)
    plsc.subcore_barrier()

    base = (c * NS + s) * EPW

    def chunk(k, carry):
        pltpu.sync_copy(dst_hbm.at[pl.ds(base + k * CH, CH)], idx_v)
        pltpu.sync_copy(ones_v, acc_sh.at[idx_v], add=True)
        return carry

    lax.fori_loop(0, N_CHUNKS, chunk, 0)
    plsc.subcore_barrier()
    sl = pl.ds(s * TILE_N, TILE_N)
    pltpu.sync_copy(acc_sh.at[sl], out_hbm.at[c, sl])


@functools.partial(
    pl.kernel,
    out_type=(
        jax.ShapeDtypeStruct((NC, N_PAD), jnp.float32),
        jax.ShapeDtypeStruct((NC, N_PAD), jnp.float32),
    ),
    mesh=_mesh,
    scratch_types=[
        pltpu.VMEM((N_PAD,), jnp.float32),
        pltpu.VMEM((CH,), jnp.int32),
        pltpu.VMEM((CH,), jnp.int32),
        pltpu.VMEM((CH,), jnp.float32),
        pltpu.VMEM_SHARED((N_PAD,), jnp.float32),
        pltpu.VMEM_SHARED((N_PAD,), jnp.float32),
    ],
    compiler_params=pltpu.CompilerParams(needs_layout_passes=False),
)
def _edgepass(src_hbm, dst_hbm, y_hbm, dis_hbm, zeros_hbm,
              a_out, c_out, table_v, gidx_v, sidx_v, vals_v, a_sh, c_sh):
    c = lax.axis_index("c")
    s = lax.axis_index("s")

    @pl.when(s == 0)
    def _():
        pltpu.sync_copy(zeros_hbm, a_sh)
        pltpu.sync_copy(zeros_hbm, c_sh)

    plsc.subcore_barrier()
    base = (c * NS + s) * EPW

    def sweep(table_hbm, gather_idx_hbm, scatter_idx_hbm, acc_sh):
        pltpu.sync_copy(table_hbm, table_v)

        def chunk(k, carry):
            off = base + k * CH
            pltpu.sync_copy(gather_idx_hbm.at[pl.ds(off, CH)], gidx_v)
            pltpu.sync_copy(scatter_idx_hbm.at[pl.ds(off, CH)], sidx_v)

            def gather16(i, carry2):
                sl = pl.ds(pl.multiple_of(i * 16, 16), 16)
                vals_v[sl] = plsc.load_gather(table_v, [gidx_v[sl]])
                return carry2

            lax.fori_loop(0, CH // 16, gather16, 0)
            pltpu.sync_copy(vals_v, acc_sh.at[sidx_v], add=True)
            return carry

        lax.fori_loop(0, N_CHUNKS, chunk, 0)

    sweep(y_hbm, src_hbm, dst_hbm, a_sh)
    sweep(dis_hbm, dst_hbm, src_hbm, c_sh)

    plsc.subcore_barrier()
    sl = pl.ds(s * TILE_N, TILE_N)
    pltpu.sync_copy(a_sh.at[sl], a_out.at[c, sl])
    pltpu.sync_copy(c_sh.at[sl], c_out.at[c, sl])


def _pw_body(cnt2_ref, x_ref, dis_ref, y_ref):
    cnt = cnt2_ref[0] + cnt2_ref[1]
    dis = lax.rsqrt(cnt + 1.0)
    dis_ref[...] = dis
    y_ref[...] = x_ref[...] * dis


_pw = pl.pallas_call(
    _pw_body,
    out_shape=(
        jax.ShapeDtypeStruct((ROWS, 128), jnp.float32),
        jax.ShapeDtypeStruct((ROWS, 128), jnp.float32),
    ),
)


def _final_body(x_ref, dis_ref, a2_ref, c2_ref, w1_ref, b1_ref, w2_ref,
                b2_ref, wo_ref, bo_ref, out_ref):
    a = a2_ref[0] + a2_ref[1]
    cc = c2_ref[0] + c2_ref[1]
    dis = dis_ref[...]
    d2 = dis * dis
    s1 = dis * a + d2 * x_ref[...]
    t = dis * cc + d2
    row = lax.broadcasted_iota(jnp.int32, (ROWS, 128), 0)
    col = lax.broadcasted_iota(jnp.int32, (ROWS, 128), 1)
    t = jnp.where(row * 128 + col < N_NODES, t, 0.0)
    pooled = b2_ref[...]
    inv_n = 1.0 / N_NODES
    for j in range(16):
        h = jnp.maximum(s1 * w1_ref[0, j] + b1_ref[0, j], 0.0)
        pooled = pooled + (jnp.sum(t * h) * inv_n) * w2_ref[pl.ds(j, 1), :]
    out_ref[...] = jnp.sum(pooled * wo_ref[...]).reshape(1, 1) + bo_ref[...]


_final = pl.pallas_call(
    _final_body,
    out_shape=jax.ShapeDtypeStruct((1, 1), jnp.float32),
)


def kernel(x, edge_index, W1, b1, W2, b2, W_out, b_out):
    src = edge_index[0].astype(jnp.int32)
    dst = edge_index[1].astype(jnp.int32)
    zeros = jnp.zeros((N_PAD,), jnp.float32)
    ones_ch = jnp.ones((CH,), jnp.float32)
    cnt2 = _hist(dst, zeros, ones_ch)
    x_pad = jnp.pad(x[:, 0], (0, N_PAD - N_NODES)).reshape(ROWS, 128)
    dis, y = _pw(cnt2.reshape(NC, ROWS, 128), x_pad)
    a2, c2 = _edgepass(src, dst, y.reshape(-1), dis.reshape(-1), zeros)
    return _final(
        x_pad, dis, a2.reshape(NC, ROWS, 128), c2.reshape(NC, ROWS, 128),
        W1, b1.reshape(1, 16), W2, b2.reshape(1, 32),
        W_out.reshape(1, 32), b_out.reshape(1, 1),
    )


# speedup vs baseline: 332.1321x; 1.2650x over previous
import functools

import jax
import jax.numpy as jnp
from jax import lax
from jax.experimental import pallas as pl
from jax.experimental.pallas import tpu as pltpu
from jax.experimental.pallas import tpu_sc as plsc

N_NODES = 100000
N_EDGES = 6400000
NC, NS = 2, 16
NW = NC * NS
N_PAD = 102400
ROWS = N_PAD // 128
EPW = N_EDGES // NW
CH = 2000
N_CHUNKS = EPW // CH
TILE_N = N_PAD // NS

_mesh = plsc.VectorSubcoreMesh(
    core_axis_name="c", subcore_axis_name="s", num_cores=NC, num_subcores=NS
)


@functools.partial(
    pl.kernel,
    out_type=jax.ShapeDtypeStruct((NC, N_PAD), jnp.float32),
    mesh=_mesh,
    scratch_types=[
        pltpu.VMEM((CH,), jnp.int32),
        pltpu.VMEM((CH,), jnp.float32),
        pltpu.VMEM_SHARED((N_PAD,), jnp.float32),
    ],
)
def _hist(dst_hbm, zeros_hbm, ones_hbm, out_hbm, idx_v, ones_v, acc_sh):
    c = lax.axis_index("c")
    s = lax.axis_index("s")

    @pl.when(s == 0)
    def _():
        pltpu.sync_copy(zeros_hbm, acc_sh)

    pltpu.sync_copy(ones_hbm, ones_v)
    plsc.subcore_barrier()

    base = (c * NS + s) * EPW

    def chunk(k, carry):
        pltpu.sync_copy(dst_hbm.at[pl.ds(base + k * CH, CH)], idx_v)
        pltpu.sync_copy(ones_v, acc_sh.at[idx_v], add=True)
        return carry

    lax.fori_loop(0, N_CHUNKS, chunk, 0)
    plsc.subcore_barrier()
    sl = pl.ds(s * TILE_N, TILE_N)
    pltpu.sync_copy(acc_sh.at[sl], out_hbm.at[c, sl])


@functools.partial(
    pl.kernel,
    out_type=(
        jax.ShapeDtypeStruct((NC, N_PAD), jnp.float32),
        jax.ShapeDtypeStruct((NC, N_PAD), jnp.float32),
    ),
    mesh=_mesh,
    scratch_types=[
        pltpu.VMEM((N_PAD,), jnp.float32),
        pltpu.VMEM((CH,), jnp.int32),
        pltpu.VMEM((CH,), jnp.int32),
        pltpu.VMEM((CH,), jnp.int32),
        pltpu.VMEM((CH,), jnp.int32),
        pltpu.VMEM((CH,), jnp.float32),
        pltpu.VMEM((CH,), jnp.float32),
        pltpu.VMEM_SHARED((N_PAD,), jnp.float32),
        pltpu.VMEM_SHARED((N_PAD,), jnp.float32),
        pltpu.SemaphoreType.DMA,
        pltpu.SemaphoreType.DMA,
        pltpu.SemaphoreType.DMA,
        pltpu.SemaphoreType.DMA,
    ],
    compiler_params=pltpu.CompilerParams(needs_layout_passes=False),
)
def _edgepass(src_hbm, dst_hbm, y_hbm, dis_hbm, zeros_hbm,
              a_out, c_out, table_v, gidx0, gidx1, sidx0, sidx1,
              vals0, vals1, a_sh, c_sh, six0, six1, ssc0, ssc1):
    c = lax.axis_index("c")
    s = lax.axis_index("s")

    @pl.when(s == 0)
    def _():
        pltpu.sync_copy(zeros_hbm, a_sh)
        pltpu.sync_copy(zeros_hbm, c_sh)

    plsc.subcore_barrier()
    base = (c * NS + s) * EPW

    def sweep(table_hbm, gather_idx_hbm, scatter_idx_hbm, acc_sh):
        pltpu.sync_copy(table_hbm, table_v)
        gbufs = (gidx0, gidx1)
        sbufs = (sidx0, sidx1)
        vbufs = (vals0, vals1)
        isems = (six0, six1)
        ssems = (ssc0, ssc1)

        def issue_idx(k, p):
            off = base + k * CH
            pltpu.async_copy(gather_idx_hbm.at[pl.ds(off, CH)], gbufs[p],
                             isems[p])
            pltpu.async_copy(scatter_idx_hbm.at[pl.ds(off, CH)], sbufs[p],
                             isems[p])

        def wait_idx(k, p):
            off = base + k * CH
            pltpu.make_async_copy(gather_idx_hbm.at[pl.ds(off, CH)], gbufs[p],
                                  isems[p]).wait()
            pltpu.make_async_copy(scatter_idx_hbm.at[pl.ds(off, CH)], sbufs[p],
                                  isems[p]).wait()

        def gather(p):
            gv, vv = gbufs[p], vbufs[p]

            def gather80(i, carry):
                for u in range(5):
                    sl = pl.ds(pl.multiple_of(i * 80 + u * 16, 16), 16)
                    vv[sl] = plsc.load_gather(table_v, [gv[sl]])
                return carry

            lax.fori_loop(0, CH // 80, gather80, 0)

        def issue_scatter(p):
            pltpu.async_copy(vbufs[p], acc_sh.at[sbufs[p]], ssems[p], add=True)

        def wait_scatter(p):
            pltpu.make_async_copy(vbufs[p], acc_sh.at[sbufs[p]],
                                  ssems[p]).wait()

        issue_idx(0, 0)

        def super_step(i, carry):
            k0 = i * 2
            wait_idx(k0, 0)
            gather(0)

            @pl.when(i > 0)
            def _():
                wait_scatter(1)

            issue_scatter(0)
            issue_idx(k0 + 1, 1)

            wait_idx(k0 + 1, 1)
            gather(1)
            wait_scatter(0)
            issue_scatter(1)

            @pl.when(i < N_CHUNKS // 2 - 1)
            def _():
                issue_idx(k0 + 2, 0)

            return carry

        lax.fori_loop(0, N_CHUNKS // 2, super_step, 0)
        wait_scatter(1)

    sweep(y_hbm, src_hbm, dst_hbm, a_sh)
    sweep(dis_hbm, dst_hbm, src_hbm, c_sh)

    plsc.subcore_barrier()
    sl = pl.ds(s * TILE_N, TILE_N)
    pltpu.sync_copy(a_sh.at[sl], a_out.at[c, sl])
    pltpu.sync_copy(c_sh.at[sl], c_out.at[c, sl])


def _pw_body(cnt2_ref, x_ref, dis_ref, y_ref):
    cnt = cnt2_ref[0] + cnt2_ref[1]
    dis = lax.rsqrt(cnt + 1.0)
    dis_ref[...] = dis
    y_ref[...] = x_ref[...] * dis


_pw = pl.pallas_call(
    _pw_body,
    out_shape=(
        jax.ShapeDtypeStruct((ROWS, 128), jnp.float32),
        jax.ShapeDtypeStruct((ROWS, 128), jnp.float32),
    ),
)


def _final_body(x_ref, dis_ref, a2_ref, c2_ref, w1_ref, b1_ref, w2_ref,
                b2_ref, wo_ref, bo_ref, out_ref):
    a = a2_ref[0] + a2_ref[1]
    cc = c2_ref[0] + c2_ref[1]
    dis = dis_ref[...]
    d2 = dis * dis
    s1 = dis * a + d2 * x_ref[...]
    t = dis * cc + d2
    row = lax.broadcasted_iota(jnp.int32, (ROWS, 128), 0)
    col = lax.broadcasted_iota(jnp.int32, (ROWS, 128), 1)
    t = jnp.where(row * 128 + col < N_NODES, t, 0.0)
    pooled = b2_ref[...]
    inv_n = 1.0 / N_NODES
    for j in range(16):
        h = jnp.maximum(s1 * w1_ref[0, j] + b1_ref[0, j], 0.0)
        pooled = pooled + (jnp.sum(t * h) * inv_n) * w2_ref[pl.ds(j, 1), :]
    out_ref[...] = jnp.sum(pooled * wo_ref[...]).reshape(1, 1) + bo_ref[...]


_final = pl.pallas_call(
    _final_body,
    out_shape=jax.ShapeDtypeStruct((1, 1), jnp.float32),
)


def kernel(x, edge_index, W1, b1, W2, b2, W_out, b_out):
    src = edge_index[0].astype(jnp.int32)
    dst = edge_index[1].astype(jnp.int32)
    zeros = jnp.zeros((N_PAD,), jnp.float32)
    ones_ch = jnp.ones((CH,), jnp.float32)
    cnt2 = _hist(dst, zeros, ones_ch)
    x_pad = jnp.pad(x[:, 0], (0, N_PAD - N_NODES)).reshape(ROWS, 128)
    dis, y = _pw(cnt2.reshape(NC, ROWS, 128), x_pad)
    a2, c2 = _edgepass(src, dst, y.reshape(-1), dis.reshape(-1), zeros)
    return _final(
        x_pad, dis, a2.reshape(NC, ROWS, 128), c2.reshape(NC, ROWS, 128),
        W1, b1.reshape(1, 16), W2, b2.reshape(1, 32),
        W_out.reshape(1, 32), b_out.reshape(1, 1),
    )


# speedup vs baseline: 347.1738x; 1.0453x over previous
import functools

import jax
import jax.numpy as jnp
from jax import lax
from jax.experimental import pallas as pl
from jax.experimental.pallas import tpu as pltpu
from jax.experimental.pallas import tpu_sc as plsc

N_NODES = 100000
N_EDGES = 6400000
NC, NS = 2, 16
NW = NC * NS
N_PAD = 102400
ROWS = N_PAD // 128
EPW = N_EDGES // NW
CH = 2000
N_CHUNKS = EPW // CH
TILE_N = N_PAD // NS

_mesh = plsc.VectorSubcoreMesh(
    core_axis_name="c", subcore_axis_name="s", num_cores=NC, num_subcores=NS
)


@functools.partial(
    pl.kernel,
    out_type=jax.ShapeDtypeStruct((NW, N_PAD), jnp.float32),
    mesh=_mesh,
    scratch_types=[
        pltpu.VMEM((N_PAD,), jnp.float32),
        pltpu.VMEM((CH,), jnp.int32),
        pltpu.VMEM((CH,), jnp.int32),
        pltpu.SemaphoreType.DMA,
        pltpu.SemaphoreType.DMA,
        pltpu.SemaphoreType.DMA,
    ],
    compiler_params=pltpu.CompilerParams(needs_layout_passes=False),
)
def _hist(dst_hbm, zeros_hbm, out_hbm, cnt_v, idx0, idx1, sz, si0, si1):
    c = lax.axis_index("c")
    s = lax.axis_index("s")
    w = c * NS + s
    base = w * EPW
    ibufs = (idx0, idx1)
    isems = (si0, si1)

    def issue_idx(k, p):
        pltpu.async_copy(dst_hbm.at[pl.ds(base + k * CH, CH)], ibufs[p],
                         isems[p])

    def wait_idx(k, p):
        pltpu.make_async_copy(dst_hbm.at[pl.ds(base + k * CH, CH)], ibufs[p],
                              isems[p]).wait()

    pltpu.async_copy(zeros_hbm, cnt_v, sz)
    issue_idx(0, 0)
    pltpu.make_async_copy(zeros_hbm, cnt_v, sz).wait()
    ones16 = jnp.ones((16,), jnp.float32)

    def count(p):
        iv = ibufs[p]

        def body(i, carry):
            for u in range(5):
                sl = pl.ds(pl.multiple_of(i * 80 + u * 16, 16), 16)
                plsc.addupdate_scatter(cnt_v, [iv[sl]], ones16)
            return carry

        lax.fori_loop(0, CH // 80, body, 0)

    def super_step(i, carry):
        k0 = i * 2
        wait_idx(k0, 0)
        issue_idx(k0 + 1, 1)
        count(0)
        wait_idx(k0 + 1, 1)

        @pl.when(i < N_CHUNKS // 2 - 1)
        def _():
            issue_idx(k0 + 2, 0)

        count(1)
        return carry

    lax.fori_loop(0, N_CHUNKS // 2, super_step, 0)
    pltpu.sync_copy(cnt_v, out_hbm.at[w])


@functools.partial(
    pl.kernel,
    out_type=(
        jax.ShapeDtypeStruct((NC, N_PAD), jnp.float32),
        jax.ShapeDtypeStruct((NC, N_PAD), jnp.float32),
    ),
    mesh=_mesh,
    scratch_types=[
        pltpu.VMEM((N_PAD,), jnp.float32),
        pltpu.VMEM((CH,), jnp.int32),
        pltpu.VMEM((CH,), jnp.int32),
        pltpu.VMEM((CH,), jnp.int32),
        pltpu.VMEM((CH,), jnp.int32),
        pltpu.VMEM((CH,), jnp.float32),
        pltpu.VMEM((CH,), jnp.float32),
        pltpu.VMEM_SHARED((N_PAD,), jnp.float32),
        pltpu.VMEM_SHARED((N_PAD,), jnp.float32),
        pltpu.SemaphoreType.DMA,
        pltpu.SemaphoreType.DMA,
        pltpu.SemaphoreType.DMA,
        pltpu.SemaphoreType.DMA,
    ],
    compiler_params=pltpu.CompilerParams(needs_layout_passes=False),
)
def _edgepass(src_hbm, dst_hbm, y_hbm, dis_hbm, zeros_hbm,
              a_out, c_out, table_v, gidx0, gidx1, sidx0, sidx1,
              vals0, vals1, a_sh, c_sh, six0, six1, ssc0, ssc1):
    c = lax.axis_index("c")
    s = lax.axis_index("s")

    @pl.when(s == 0)
    def _():
        pltpu.sync_copy(zeros_hbm, a_sh)
        pltpu.sync_copy(zeros_hbm, c_sh)

    plsc.subcore_barrier()
    base = (c * NS + s) * EPW

    def sweep(table_hbm, gather_idx_hbm, scatter_idx_hbm, acc_sh):
        pltpu.sync_copy(table_hbm, table_v)
        gbufs = (gidx0, gidx1)
        sbufs = (sidx0, sidx1)
        vbufs = (vals0, vals1)
        isems = (six0, six1)
        ssems = (ssc0, ssc1)

        def issue_idx(k, p):
            off = base + k * CH
            pltpu.async_copy(gather_idx_hbm.at[pl.ds(off, CH)], gbufs[p],
                             isems[p])
            pltpu.async_copy(scatter_idx_hbm.at[pl.ds(off, CH)], sbufs[p],
                             isems[p])

        def wait_idx(k, p):
            off = base + k * CH
            pltpu.make_async_copy(gather_idx_hbm.at[pl.ds(off, CH)], gbufs[p],
                                  isems[p]).wait()
            pltpu.make_async_copy(scatter_idx_hbm.at[pl.ds(off, CH)], sbufs[p],
                                  isems[p]).wait()

        def gather(p):
            gv, vv = gbufs[p], vbufs[p]

            def gather80(i, carry):
                for u in range(5):
                    sl = pl.ds(pl.multiple_of(i * 80 + u * 16, 16), 16)
                    vv[sl] = plsc.load_gather(table_v, [gv[sl]])
                return carry

            lax.fori_loop(0, CH // 80, gather80, 0)

        def issue_scatter(p):
            pltpu.async_copy(vbufs[p], acc_sh.at[sbufs[p]], ssems[p], add=True)

        def wait_scatter(p):
            pltpu.make_async_copy(vbufs[p], acc_sh.at[sbufs[p]],
                                  ssems[p]).wait()

        issue_idx(0, 0)

        def super_step(i, carry):
            k0 = i * 2
            wait_idx(k0, 0)
            gather(0)

            @pl.when(i > 0)
            def _():
                wait_scatter(1)

            issue_scatter(0)
            issue_idx(k0 + 1, 1)

            wait_idx(k0 + 1, 1)
            gather(1)
            wait_scatter(0)
            issue_scatter(1)

            @pl.when(i < N_CHUNKS // 2 - 1)
            def _():
                issue_idx(k0 + 2, 0)

            return carry

        lax.fori_loop(0, N_CHUNKS // 2, super_step, 0)
        wait_scatter(1)

    sweep(y_hbm, src_hbm, dst_hbm, a_sh)
    sweep(dis_hbm, dst_hbm, src_hbm, c_sh)

    plsc.subcore_barrier()
    sl = pl.ds(s * TILE_N, TILE_N)
    pltpu.sync_copy(a_sh.at[sl], a_out.at[c, sl])
    pltpu.sync_copy(c_sh.at[sl], c_out.at[c, sl])


PW_BLK = ROWS // 4


def _pw_body(cntw_ref, x_ref, dis_ref, y_ref):
    cnt = jnp.sum(cntw_ref[...], axis=0)
    dis = lax.rsqrt(cnt + 1.0)
    dis_ref[...] = dis
    y_ref[...] = x_ref[...] * dis


_pw = pl.pallas_call(
    _pw_body,
    grid=(4,),
    in_specs=[
        pl.BlockSpec((NW, PW_BLK, 128), lambda i: (0, i, 0)),
        pl.BlockSpec((PW_BLK, 128), lambda i: (i, 0)),
    ],
    out_specs=[
        pl.BlockSpec((PW_BLK, 128), lambda i: (i, 0)),
        pl.BlockSpec((PW_BLK, 128), lambda i: (i, 0)),
    ],
    out_shape=(
        jax.ShapeDtypeStruct((ROWS, 128), jnp.float32),
        jax.ShapeDtypeStruct((ROWS, 128), jnp.float32),
    ),
)


def _final_body(x_ref, dis_ref, a2_ref, c2_ref, w1_ref, b1_ref, w2_ref,
                b2_ref, wo_ref, bo_ref, out_ref):
    a = a2_ref[0] + a2_ref[1]
    cc = c2_ref[0] + c2_ref[1]
    dis = dis_ref[...]
    d2 = dis * dis
    s1 = dis * a + d2 * x_ref[...]
    t = dis * cc + d2
    row = lax.broadcasted_iota(jnp.int32, (ROWS, 128), 0)
    col = lax.broadcasted_iota(jnp.int32, (ROWS, 128), 1)
    t = jnp.where(row * 128 + col < N_NODES, t, 0.0)
    pooled = b2_ref[...]
    inv_n = 1.0 / N_NODES
    for j in range(16):
        h = jnp.maximum(s1 * w1_ref[0, j] + b1_ref[0, j], 0.0)
        pooled = pooled + (jnp.sum(t * h) * inv_n) * w2_ref[pl.ds(j, 1), :]
    out_ref[...] = jnp.sum(pooled * wo_ref[...]).reshape(1, 1) + bo_ref[...]


_final = pl.pallas_call(
    _final_body,
    out_shape=jax.ShapeDtypeStruct((1, 1), jnp.float32),
)


def kernel(x, edge_index, W1, b1, W2, b2, W_out, b_out):
    src = edge_index[0].astype(jnp.int32)
    dst = edge_index[1].astype(jnp.int32)
    zeros = jnp.zeros((N_PAD,), jnp.float32)
    cntw = _hist(dst, zeros)
    x_pad = jnp.pad(x[:, 0], (0, N_PAD - N_NODES)).reshape(ROWS, 128)
    dis, y = _pw(cntw.reshape(NW, ROWS, 128), x_pad)
    a2, c2 = _edgepass(src, dst, y.reshape(-1), dis.reshape(-1), zeros)
    return _final(
        x_pad, dis, a2.reshape(NC, ROWS, 128), c2.reshape(NC, ROWS, 128),
        W1, b1.reshape(1, 16), W2, b2.reshape(1, 32),
        W_out.reshape(1, 32), b_out.reshape(1, 1),
    )
